# baseline (speedup 1.0000x reference)
import functools

import jax
import jax.numpy as jnp
from jax import lax
from jax.experimental import pallas as pl
from jax.experimental.pallas import tpu as pltpu
from jax.experimental.pallas import tpu_sc as plsc


@functools.cache
def _make_gather(V, D, B, L):
    info = plsc.get_sparse_core_info()
    NC, NS = info.num_cores, info.num_subcores
    NW = NC * NS
    assert B % NW == 0
    CB = B // NW

    mesh = plsc.VectorSubcoreMesh(core_axis_name="c", subcore_axis_name="s")

    @functools.partial(
        pl.kernel,
        mesh=mesh,
        out_type=jax.ShapeDtypeStruct((L, B, D), jnp.float32),
        compiler_params=pltpu.CompilerParams(use_tc_tiling_on_sc=False),
        scratch_types=[
            pltpu.VMEM((L, CB), jnp.int32),
            pltpu.VMEM((2, CB, D), jnp.float32),
            pltpu.SemaphoreType.DMA,
            pltpu.SemaphoreType.DMA,
        ],
    )
    def gather_kernel(table_hbm, idxt_hbm, out_hbm, idx_v, rows_v, gsem, wsem):
        wid = lax.axis_index("s") * NC + lax.axis_index("c")
        c0 = wid * CB
        pltpu.sync_copy(idxt_hbm.at[:, pl.ds(c0, CB)], idx_v)

        def gather(l, slot):
            return pltpu.async_copy(
                table_hbm.at[idx_v.at[l]], rows_v.at[slot], gsem
            )

        def gather_wait(slot):
            pltpu.make_async_copy(
                table_hbm.at[idx_v.at[0]], rows_v.at[slot], gsem
            ).wait()

        def write(l, slot):
            return pltpu.async_copy(
                rows_v.at[slot], out_hbm.at[l, pl.ds(c0, CB)], wsem
            )

        def write_wait(slot):
            pltpu.make_async_copy(
                rows_v.at[slot], out_hbm.at[0, pl.ds(c0, CB)], wsem
            ).wait()

        gather(0, 0)

        def body(l, carry):
            cur = lax.rem(l, 2)

            gather_wait(cur)

            @pl.when(l >= 1)
            def _():
                write_wait(1 - cur)

            @pl.when(l + 1 < L)
            def _():
                gather(l + 1, 1 - cur)

            write(l, cur)
            return carry

        lax.fori_loop(0, L, body, 0)
        write_wait((L - 1) % 2)

    return gather_kernel


def kernel(word_table, indices):
    B, L = indices.shape
    V, D = word_table.shape
    fn = _make_gather(V, D, B, L)
    out_t = fn(word_table, indices.T)
    return out_t.transpose(1, 0, 2)
